# baseline (speedup 1.0000x reference)
import functools

import jax
import jax.numpy as jnp
from jax import lax
from jax.experimental import pallas as pl
from jax.experimental.pallas import tpu as pltpu
from jax.experimental.pallas import tpu_sc as plsc

VOCAB = 1000000
EMBED = 64
NTOKENS = 200
BATCH = 4096

TOTAL = BATCH * NTOKENS
NUM_WORKERS = 32
PER_WORKER = TOTAL // NUM_WORKERS
CHUNK = NTOKENS
NCHUNKS = PER_WORKER // CHUNK
GROUPS = [16] * 12 + [8]
ADD_SPLITS = [15] * 12 + [20]

_mesh = plsc.VectorSubcoreMesh(core_axis_name="c", subcore_axis_name="s")


@functools.partial(
    pl.kernel,
    mesh=_mesh,
    out_type=jax.ShapeDtypeStruct((TOTAL, EMBED), jnp.float32),
    scratch_types=[
        pltpu.VMEM((CHUNK + 8,), jnp.int32),
        pltpu.VMEM((CHUNK + 8,), jnp.int32),
        pltpu.VMEM((CHUNK, EMBED), jnp.float32),
        pltpu.VMEM((CHUNK, EMBED), jnp.float32),
        pltpu.VMEM((CHUNK, EMBED), jnp.float32),
        pltpu.SemaphoreType.DMA,
        pltpu.SemaphoreType.DMA,
        pltpu.SemaphoreType.DMA,
        pltpu.SemaphoreType.DMA,
        pltpu.SemaphoreType.DMA,
        pltpu.SemaphoreType.DMA,
    ],
    compiler_params=pltpu.CompilerParams(use_tc_tiling_on_sc=True),
)
def _embed_sc(tokens_hbm, table_hbm, pos_hbm, out_hbm,
              idx_a, idx_b, rows_a, rows_b, pos_v,
              sem_ia, sem_ib, sem_ra, sem_rb, sem_oa, sem_ob):
    wid = lax.axis_index("s") * 2 + lax.axis_index("c")
    base = wid * PER_WORKER
    last = NCHUNKS - 1

    pltpu.sync_copy(pos_hbm, pos_v)

    def extract_group(idx_v, j, gsz):
        v = idx_v[pl.ds(j, 16)]
        return [v[i] for i in range(gsz)]

    def enqueue_group(rows_v, sem, j, ts):
        for i, t in enumerate(ts):
            pltpu.async_copy(
                table_hbm.at[lax.shift_right_logical(t, 3),
                             pl.ds(lax.bitwise_and(t, 7), 1)],
                rows_v.at[pl.ds(j + i, 1)], sem)

    def add_slice(rows_v, r0, nr):
        def row_body(r, c2):
            for cc in range(EMBED // 16):
                sl = pl.ds(cc * 16, 16)
                rows_v[r, sl] = rows_v[r, sl] + pos_v[r, sl]
            return c2

        lax.fori_loop(r0, r0 + nr, row_body, 0)

    def fire_interleaved(idx_f, rows_f, sem_f, rows_add):
        offs = [0]
        for gsz in GROUPS:
            offs.append(offs[-1] + gsz)
        ts = extract_group(idx_f, 0, GROUPS[0])
        r0 = 0
        for k, (gsz, nr) in enumerate(zip(GROUPS, ADD_SPLITS)):
            ts_next = (extract_group(idx_f, offs[k + 1], GROUPS[k + 1])
                       if k + 1 < len(GROUPS) else None)
            enqueue_group(rows_f, sem_f, offs[k], ts)
            add_slice(rows_add, r0, nr)
            ts = ts_next
            r0 += nr

    def fire_all(idx_v, rows_v, sem):
        j = 0
        for gsz in GROUPS:
            enqueue_group(rows_v, sem, j, extract_group(idx_v, j, gsz))
            j += gsz

    def drain_rows(rows_v, sem):
        pltpu.make_async_copy(
            table_hbm.at[0, pl.ds(0, 1)], rows_v.at[pl.ds(0, CHUNK)], sem
        ).wait()

    def idx_fetch(c, idx_v, sem):
        c = jnp.minimum(c, last)
        pltpu.async_copy(tokens_hbm.at[pl.ds(base + c * CHUNK, CHUNK)],
                         idx_v.at[pl.ds(0, CHUNK)], sem)

    def idx_wait(idx_v, sem):
        pltpu.make_async_copy(tokens_hbm.at[pl.ds(0, CHUNK)],
                              idx_v.at[pl.ds(0, CHUNK)], sem).wait()

    def out_start(rows_v, c, sem):
        pltpu.async_copy(rows_v, out_hbm.at[pl.ds(base + c * CHUNK, CHUNK)], sem)

    def out_wait(rows_v, sem):
        pltpu.make_async_copy(rows_v, out_hbm.at[pl.ds(0, CHUNK)], sem).wait()

    pltpu.sync_copy(tokens_hbm.at[pl.ds(base, CHUNK)], idx_a.at[pl.ds(0, CHUNK)])
    fire_all(idx_a, rows_a, sem_ra)
    idx_fetch(1, idx_b, sem_ib)

    def pair_body(g, carry):
        ca = 2 * g

        idx_wait(idx_b, sem_ib)

        @pl.when(g > 0)
        def _():
            out_wait(rows_b, sem_ob)

        idx_fetch(ca + 2, idx_a, sem_ia)
        drain_rows(rows_a, sem_ra)
        fire_interleaved(idx_b, rows_b, sem_rb, rows_a)
        out_start(rows_a, ca, sem_oa)

        idx_wait(idx_a, sem_ia)
        out_wait(rows_a, sem_oa)
        idx_fetch(ca + 3, idx_b, sem_ib)
        drain_rows(rows_b, sem_rb)
        fire_interleaved(idx_a, rows_a, sem_ra, rows_b)
        out_start(rows_b, ca + 1, sem_ob)
        return carry

    lax.fori_loop(0, NCHUNKS // 2, pair_body, 0)

    idx_wait(idx_b, sem_ib)
    drain_rows(rows_a, sem_ra)
    out_wait(rows_b, sem_ob)


def kernel(tokens, input_embedding, position_embedding):
    flat = tokens.reshape(-1).astype(jnp.int32)
    table3 = input_embedding.reshape(VOCAB // 8, 8, EMBED)
    out = _embed_sc(flat, table3, position_embedding)
    return out.reshape(BATCH, NTOKENS, EMBED)
